# baseline (speedup 1.0000x reference)
import functools

import jax
import jax.numpy as jnp
from jax import lax
from jax.experimental import pallas as pl
from jax.experimental.pallas import tpu as pltpu
from jax.experimental.pallas import tpu_sc as plsc

N_POINTS = 8192
FEAT_DIM = 512
NUM_LABELS = 200
NUM_NEG = 32
LPAD = 256

POS_THRESH = 0.1
NEG_THRESH = 0.5
EPS = 1e-07

BLK = 1024
GRID = N_POINTS // BLK

NW = 32
ROWS = N_POINTS // NW
GROUPS = ROWS // 16
CHUNKS = 4
CROWS = ROWS // CHUNKS
HIMASK = -65536


def _dist_body(x_ref, a_ref, s_ref):
    f = x_ref[...]
    a = a_ref[...]
    g = lax.dot_general(f, a, (((1,), (1,)), ((), ())),
                        preferred_element_type=jnp.float32)
    fn = jnp.sum(f * f, axis=1, keepdims=True)
    ones = jnp.ones((1, FEAT_DIM), jnp.float32)
    an = lax.dot_general(ones, a * a, (((1,), (1,)), ((), ())),
                         preferred_element_type=jnp.float32)
    d2 = jnp.maximum(fn + an - 2.0 * g, 0.0)
    s = jnp.sqrt(d2 + EPS)
    lo = lax.shift_right_logical(
        lax.bitcast_convert_type(s[:, :128], jnp.int32), 16)
    hi = lax.bitcast_convert_type(s[:, 128:], jnp.int32) & HIMASK
    s_ref[...] = lo | hi


_sc_mesh = plsc.VectorSubcoreMesh(core_axis_name="c", subcore_axis_name="s")


@functools.partial(
    pl.kernel,
    mesh=_sc_mesh,
    compiler_params=pltpu.CompilerParams(
        use_tc_tiling_on_sc=False, needs_layout_passes=False),
    out_type=[
        jax.ShapeDtypeStruct((N_POINTS,), jnp.float32),
        jax.ShapeDtypeStruct((N_POINTS,), jnp.float32),
        jax.ShapeDtypeStruct((NW, 32), jnp.float32),
    ],
    scratch_types=[
        pltpu.VMEM((ROWS, 128), jnp.int32),
        pltpu.VMEM((ROWS,), jnp.int32),
        pltpu.VMEM((ROWS, NUM_NEG), jnp.int32),
        pltpu.VMEM((ROWS,), jnp.float32),
        pltpu.VMEM((ROWS,), jnp.float32),
        pltpu.VMEM((32,), jnp.float32),
        [pltpu.SemaphoreType.DMA] * CHUNKS,
    ],
)
def _sc_gather(s_hbm, lab_hbm, neg_hbm, pos_hbm, negout_hbm,
               part_hbm, s_v, lab_v, neg_v, pos_v, nout_v, psum_v, sems):
    wid = lax.axis_index("s") * 2 + lax.axis_index("c")
    base = wid * ROWS
    pltpu.sync_copy(lab_hbm.at[pl.ds(base, ROWS)], lab_v)
    pltpu.sync_copy(neg_hbm.at[pl.ds(base, ROWS)], neg_v)
    copies = [
        pltpu.async_copy(
            s_hbm.at[pl.ds(base + c * CROWS, CROWS)],
            s_v.at[pl.ds(c * CROWS, CROWS)],
            sems[c])
        for c in range(CHUNKS)
    ]

    def unpack(word, col):
        bits = jnp.where(col < 128, word << 16, word & HIMASK)
        return plsc.bitcast(bits, jnp.float32)

    pos_sum = jnp.zeros((16,), jnp.float32)
    neg_sum = jnp.zeros((16,), jnp.float32)
    for c in range(CHUNKS):
        copies[c].wait()
        for g in range(c * GROUPS // CHUNKS, (c + 1) * GROUPS // CHUNKS):
            rows = lax.iota(jnp.int32, 16) + g * 16
            lab = lab_v[pl.ds(g * 16, 16)]
            dpos = unpack(plsc.load_gather(s_v, [rows, lab & 127]), lab)
            pos_val = jnp.maximum(dpos - POS_THRESH, 0.0)
            pos_v[pl.ds(g * 16, 16)] = pos_val
            pos_sum = pos_sum + pos_val
            acc = jnp.zeros((16,), jnp.float32)
            for k in range(NUM_NEG):
                nk = plsc.load_gather(
                    neg_v, [rows, jnp.full((16,), k, jnp.int32)])
                acc = acc + unpack(
                    plsc.load_gather(s_v, [rows, nk & 127]), nk)
            neg_val = jnp.maximum(NEG_THRESH - acc * (1.0 / NUM_NEG), 0.0)
            nout_v[pl.ds(g * 16, 16)] = neg_val
            neg_sum = neg_sum + neg_val

    psum_v[pl.ds(0, 16)] = pos_sum
    psum_v[pl.ds(16, 16)] = neg_sum
    pltpu.sync_copy(pos_v, pos_hbm.at[pl.ds(base, ROWS)])
    pltpu.sync_copy(nout_v, negout_hbm.at[pl.ds(base, ROWS)])
    pltpu.sync_copy(psum_v, part_hbm.at[wid])


@jax.jit
def _run(features, labels, anchor_feats, neg_inds):
    a_pad = jnp.zeros((LPAD, FEAT_DIM), jnp.float32).at[:NUM_LABELS].set(
        anchor_feats)
    packed = pl.pallas_call(
        _dist_body,
        grid=(GRID,),
        in_specs=[
            pl.BlockSpec((BLK, FEAT_DIM), lambda i: (i, 0)),
            pl.BlockSpec((LPAD, FEAT_DIM), lambda i: (0, 0)),
        ],
        out_specs=pl.BlockSpec((BLK, 128), lambda i: (i, 0)),
        out_shape=jax.ShapeDtypeStruct((N_POINTS, 128), jnp.int32),
    )(features, a_pad)
    pos, neg, part = _sc_gather(packed, labels, neg_inds)
    loss = part.sum() * (1.0 / N_POINTS)
    return (loss, pos, neg)


def kernel(features, labels, anchor_feats, neg_inds):
    return _run(features, labels, anchor_feats, neg_inds)


# speedup vs baseline: 1.1749x; 1.1749x over previous
import functools

import jax
import jax.numpy as jnp
from jax import lax
from jax.experimental import pallas as pl
from jax.experimental.pallas import tpu as pltpu
from jax.experimental.pallas import tpu_sc as plsc

N_POINTS = 8192
FEAT_DIM = 512
NUM_LABELS = 200
NUM_NEG = 32

POS_THRESH = 0.1
NEG_THRESH = 0.5
EPS = 1e-07

BLK = 1024
GRID = N_POINTS // BLK

NW = 32
ROWS = N_POINTS // NW
GROUPS = ROWS // 16
CHUNKS = 2
CROWS = ROWS // CHUNKS
GPC = GROUPS // CHUNKS
HIMASK = -65536
RND = 32768


def _dist_body(x_ref, a_ref, s_ref):
    f = x_ref[...]
    a = a_ref[...]
    g = lax.dot_general(f, a, (((1,), (1,)), ((), ())),
                        preferred_element_type=jnp.float32)
    fn = jnp.sum(f * f, axis=1, keepdims=True)
    ones = jnp.ones((1, FEAT_DIM), jnp.float32)
    an = lax.dot_general(ones, a * a, (((1,), (1,)), ((), ())),
                         preferred_element_type=jnp.float32)
    d2 = jnp.maximum(fn + an - 2.0 * g, 0.0)
    s = jnp.sqrt(d2 + EPS)
    lo = lax.shift_right_logical(
        lax.bitcast_convert_type(s[:, :128], jnp.int32) + RND, 16)
    s_ref[...] = lo
    hi = (lax.bitcast_convert_type(s[:, 128:], jnp.int32) + RND) & HIMASK
    s_ref[:, :72] = s_ref[:, :72] | hi


_sc_mesh = plsc.VectorSubcoreMesh(core_axis_name="c", subcore_axis_name="s")


@functools.partial(
    pl.kernel,
    mesh=_sc_mesh,
    compiler_params=pltpu.CompilerParams(
        use_tc_tiling_on_sc=False, needs_layout_passes=False),
    out_type=[
        jax.ShapeDtypeStruct((N_POINTS,), jnp.float32),
        jax.ShapeDtypeStruct((N_POINTS,), jnp.float32),
        jax.ShapeDtypeStruct((NW, 32), jnp.float32),
    ],
    scratch_types=[
        pltpu.VMEM((ROWS, 128), jnp.int32),
        pltpu.VMEM((ROWS,), jnp.int32),
        pltpu.VMEM((NUM_NEG, ROWS), jnp.int32),
        pltpu.VMEM((ROWS,), jnp.float32),
        pltpu.VMEM((ROWS,), jnp.float32),
        pltpu.VMEM((32,), jnp.float32),
        [pltpu.SemaphoreType.DMA] * CHUNKS,
    ],
)
def _sc_gather(s_hbm, lab_hbm, negt_hbm, pos_hbm, negout_hbm,
               part_hbm, s_v, lab_v, neg_v, pos_v, nout_v, psum_v, sems):
    wid = lax.axis_index("s") * 2 + lax.axis_index("c")
    base = wid * ROWS
    pltpu.sync_copy(lab_hbm.at[pl.ds(base, ROWS)], lab_v)
    pltpu.sync_copy(negt_hbm.at[:, pl.ds(base, ROWS)], neg_v)
    copies = [
        pltpu.async_copy(
            s_hbm.at[pl.ds(base + c * CROWS, CROWS)],
            s_v.at[pl.ds(c * CROWS, CROWS)],
            sems[c])
        for c in range(CHUNKS)
    ]

    def unpack(word, col):
        bits = jnp.where(col < 128, word << 16, word & HIMASK)
        return plsc.bitcast(bits, jnp.float32)

    def group(g, carry):
        pos_acc, neg_acc = carry
        rows = lax.iota(jnp.int32, 16) + g * 16
        lab = lab_v[pl.ds(g * 16, 16)]
        dpos = unpack(plsc.load_gather(s_v, [rows, lab & 127]), lab)
        pos_val = jnp.maximum(dpos - POS_THRESH, 0.0)
        pos_v[pl.ds(g * 16, 16)] = pos_val
        acc = jnp.zeros((16,), jnp.float32)
        for k in range(NUM_NEG):
            nk = neg_v[k, pl.ds(g * 16, 16)]
            acc = acc + unpack(plsc.load_gather(s_v, [rows, nk & 127]), nk)
        neg_val = jnp.maximum(NEG_THRESH - acc * (1.0 / NUM_NEG), 0.0)
        nout_v[pl.ds(g * 16, 16)] = neg_val
        return pos_acc + pos_val, neg_acc + neg_val

    pos_sum = jnp.zeros((16,), jnp.float32)
    neg_sum = jnp.zeros((16,), jnp.float32)
    for c in range(CHUNKS):
        copies[c].wait()
        pos_sum, neg_sum = lax.fori_loop(
            c * GPC, (c + 1) * GPC, group, (pos_sum, neg_sum))

    psum_v[pl.ds(0, 16)] = pos_sum
    psum_v[pl.ds(16, 16)] = neg_sum
    pltpu.sync_copy(pos_v, pos_hbm.at[pl.ds(base, ROWS)])
    pltpu.sync_copy(nout_v, negout_hbm.at[pl.ds(base, ROWS)])
    pltpu.sync_copy(psum_v, part_hbm.at[wid])


@jax.jit
def _run(features, labels, anchor_feats, neg_inds):
    packed = pl.pallas_call(
        _dist_body,
        grid=(GRID,),
        in_specs=[
            pl.BlockSpec((BLK, FEAT_DIM), lambda i: (i, 0)),
            pl.BlockSpec((NUM_LABELS, FEAT_DIM), lambda i: (0, 0)),
        ],
        out_specs=pl.BlockSpec((BLK, 128), lambda i: (i, 0)),
        out_shape=jax.ShapeDtypeStruct((N_POINTS, 128), jnp.int32),
    )(features, anchor_feats)
    pos, neg, part = _sc_gather(packed, labels, neg_inds.T)
    loss = part.sum() * (1.0 / N_POINTS)
    return (loss, pos, neg)


def kernel(features, labels, anchor_feats, neg_inds):
    return _run(features, labels, anchor_feats, neg_inds)


# speedup vs baseline: 1.2144x; 1.0335x over previous
import functools

import jax
import jax.numpy as jnp
from jax import lax
from jax.experimental import pallas as pl
from jax.experimental.pallas import tpu as pltpu
from jax.experimental.pallas import tpu_sc as plsc

N_POINTS = 8192
FEAT_DIM = 512
NUM_LABELS = 200
NUM_NEG = 32

POS_THRESH = 0.1
NEG_THRESH = 0.5
EPS = 1e-07

BLK = 2048
GRID = N_POINTS // BLK

NW = 32
ROWS = N_POINTS // NW
GROUPS = ROWS // 16
CHUNKS = 2
CROWS = ROWS // CHUNKS
GPC = GROUPS // CHUNKS
HIMASK = -65536
RND = 32768


def _dist_body(x_ref, a_ref, s_ref):
    f = x_ref[...]
    a = a_ref[...]
    g = lax.dot_general(f, a, (((1,), (1,)), ((), ())),
                        preferred_element_type=jnp.float32)
    fn = jnp.sum(f * f, axis=1, keepdims=True)
    ones = jnp.ones((1, FEAT_DIM), jnp.float32)
    an = lax.dot_general(ones, a * a, (((1,), (1,)), ((), ())),
                         preferred_element_type=jnp.float32)
    d2 = jnp.maximum(fn + an - 2.0 * g, 0.0)
    s = jnp.sqrt(d2 + EPS)
    lo = lax.shift_right_logical(
        lax.bitcast_convert_type(s[:, :128], jnp.int32) + RND, 16)
    s_ref[...] = lo
    hi = (lax.bitcast_convert_type(s[:, 128:], jnp.int32) + RND) & HIMASK
    s_ref[:, :72] = s_ref[:, :72] | hi


_sc_mesh = plsc.VectorSubcoreMesh(core_axis_name="c", subcore_axis_name="s")


@functools.partial(
    pl.kernel,
    mesh=_sc_mesh,
    compiler_params=pltpu.CompilerParams(
        use_tc_tiling_on_sc=False, needs_layout_passes=False),
    out_type=[
        jax.ShapeDtypeStruct((N_POINTS,), jnp.float32),
        jax.ShapeDtypeStruct((N_POINTS,), jnp.float32),
        jax.ShapeDtypeStruct((NW, 32), jnp.float32),
    ],
    scratch_types=[
        pltpu.VMEM((ROWS, 128), jnp.int32),
        pltpu.VMEM((ROWS,), jnp.int32),
        pltpu.VMEM((NUM_NEG, CHUNKS, 128), jnp.int32),
        pltpu.VMEM((ROWS,), jnp.float32),
        pltpu.VMEM((ROWS,), jnp.float32),
        pltpu.VMEM((32,), jnp.float32),
        [pltpu.SemaphoreType.DMA] * CHUNKS,
    ],
)
def _sc_gather(s_hbm, lab_hbm, negt_hbm, pos_hbm, negout_hbm,
               part_hbm, s_v, lab_v, neg_v, pos_v, nout_v, psum_v, sems):
    wid = lax.axis_index("s") * 2 + lax.axis_index("c")
    base = wid * ROWS
    pltpu.sync_copy(lab_hbm.at[pl.ds(base, ROWS)], lab_v)
    pltpu.sync_copy(negt_hbm.at[:, pl.ds(wid * CHUNKS, CHUNKS), :], neg_v)
    copies = [
        pltpu.async_copy(
            s_hbm.at[pl.ds(base + c * CROWS, CROWS)],
            s_v.at[pl.ds(c * CROWS, CROWS)],
            sems[c])
        for c in range(CHUNKS)
    ]

    def unpack(word, col):
        bits = jnp.where(col < 128, word << 16, word & HIMASK)
        return plsc.bitcast(bits, jnp.float32)

    def make_group(c):
        def group(j, carry):
            pos_acc, neg_acc = carry
            g = c * GPC + j
            rows = lax.iota(jnp.int32, 16) + g * 16
            lab = lab_v[pl.ds(g * 16, 16)]
            dpos = unpack(plsc.load_gather(s_v, [rows, lab & 127]), lab)
            pos_val = jnp.maximum(dpos - POS_THRESH, 0.0)
            pos_v[pl.ds(g * 16, 16)] = pos_val
            acc = jnp.zeros((16,), jnp.float32)
            for k in range(NUM_NEG):
                nk = neg_v[k, c, pl.ds(j * 16, 16)]
                acc = acc + unpack(
                    plsc.load_gather(s_v, [rows, nk & 127]), nk)
            neg_val = jnp.maximum(NEG_THRESH - acc * (1.0 / NUM_NEG), 0.0)
            nout_v[pl.ds(g * 16, 16)] = neg_val
            return pos_acc + pos_val, neg_acc + neg_val
        return group

    pos_sum = jnp.zeros((16,), jnp.float32)
    neg_sum = jnp.zeros((16,), jnp.float32)
    for c in range(CHUNKS):
        copies[c].wait()
        pos_sum, neg_sum = lax.fori_loop(
            0, GPC, make_group(c), (pos_sum, neg_sum))

    psum_v[pl.ds(0, 16)] = pos_sum
    psum_v[pl.ds(16, 16)] = neg_sum
    pltpu.sync_copy(pos_v, pos_hbm.at[pl.ds(base, ROWS)])
    pltpu.sync_copy(nout_v, negout_hbm.at[pl.ds(base, ROWS)])
    pltpu.sync_copy(psum_v, part_hbm.at[wid])


@jax.jit
def _run(features, labels, anchor_feats, neg_inds):
    packed = pl.pallas_call(
        _dist_body,
        grid=(GRID,),
        in_specs=[
            pl.BlockSpec((BLK, FEAT_DIM), lambda i: (i, 0)),
            pl.BlockSpec((NUM_LABELS, FEAT_DIM), lambda i: (0, 0)),
        ],
        out_specs=pl.BlockSpec((BLK, 128), lambda i: (i, 0)),
        out_shape=jax.ShapeDtypeStruct((N_POINTS, 128), jnp.int32),
    )(features, anchor_feats)
    negl = neg_inds.T.reshape(NUM_NEG, N_POINTS // 128, 128)
    pos, neg, part = _sc_gather(packed, labels, negl)
    loss = part.sum() * (1.0 / N_POINTS)
    return (loss, pos, neg)


def kernel(features, labels, anchor_feats, neg_inds):
    return _run(features, labels, anchor_feats, neg_inds)


# speedup vs baseline: 1.2450x; 1.0252x over previous
import functools

import jax
import jax.numpy as jnp
from jax import lax
from jax.experimental import pallas as pl
from jax.experimental.pallas import tpu as pltpu
from jax.experimental.pallas import tpu_sc as plsc

N_POINTS = 8192
FEAT_DIM = 512
NUM_LABELS = 200
NUM_NEG = 32

POS_THRESH = 0.1
NEG_THRESH = 0.5
EPS = 1e-07

BLK = 2048
GRID = N_POINTS // BLK

NW = 32
ROWS = N_POINTS // NW
GROUPS = ROWS // 16
CHUNKS = 4
CROWS = ROWS // CHUNKS
GPC = GROUPS // CHUNKS
NBLK = ROWS // 128
HIMASK = -65536
RND = 32768


def _dist_body(x_ref, a_ref, s_ref):
    f = x_ref[...]
    a = a_ref[...]
    g = lax.dot_general(f, a, (((1,), (1,)), ((), ())),
                        preferred_element_type=jnp.float32)
    fn = jnp.sum(f * f, axis=1, keepdims=True)
    ones = jnp.ones((1, FEAT_DIM), jnp.float32)
    an = lax.dot_general(ones, a * a, (((1,), (1,)), ((), ())),
                         preferred_element_type=jnp.float32)
    d2 = jnp.maximum(fn + an - 2.0 * g, 0.0)
    s = jnp.sqrt(d2 + EPS)
    lo = lax.shift_right_logical(
        lax.bitcast_convert_type(s[:, :128], jnp.int32) + RND, 16)
    s_ref[...] = lo
    hi = (lax.bitcast_convert_type(s[:, 128:], jnp.int32) + RND) & HIMASK
    s_ref[:, :72] = s_ref[:, :72] | hi


_sc_mesh = plsc.VectorSubcoreMesh(core_axis_name="c", subcore_axis_name="s")


@functools.partial(
    pl.kernel,
    mesh=_sc_mesh,
    compiler_params=pltpu.CompilerParams(
        use_tc_tiling_on_sc=False, needs_layout_passes=False),
    out_type=[
        jax.ShapeDtypeStruct((N_POINTS,), jnp.float32),
        jax.ShapeDtypeStruct((N_POINTS,), jnp.float32),
        jax.ShapeDtypeStruct((NW, 32), jnp.float32),
    ],
    scratch_types=[
        pltpu.VMEM((ROWS, 128), jnp.int32),
        pltpu.VMEM((ROWS,), jnp.int32),
        pltpu.VMEM((NUM_NEG, NBLK, 128), jnp.int32),
        pltpu.VMEM((ROWS,), jnp.float32),
        pltpu.VMEM((ROWS,), jnp.float32),
        pltpu.VMEM((32,), jnp.float32),
        [pltpu.SemaphoreType.DMA] * (CHUNKS + 2),
    ],
)
def _sc_gather(s_hbm, lab_hbm, negt_hbm, pos_hbm, negout_hbm,
               part_hbm, s_v, lab_v, neg_v, pos_v, nout_v, psum_v, sems):
    wid = lax.axis_index("s") * 2 + lax.axis_index("c")
    base = wid * ROWS
    lab_cp = pltpu.async_copy(
        lab_hbm.at[pl.ds(base, ROWS)], lab_v, sems[CHUNKS])
    neg_cp = pltpu.async_copy(
        negt_hbm.at[:, pl.ds(wid * NBLK, NBLK), :], neg_v, sems[CHUNKS + 1])
    copies = [
        pltpu.async_copy(
            s_hbm.at[pl.ds(base + c * CROWS, CROWS)],
            s_v.at[pl.ds(c * CROWS, CROWS)],
            sems[c])
        for c in range(CHUNKS)
    ]
    lab_cp.wait()
    neg_cp.wait()

    def unpack(word, col):
        bits = jnp.where(col < 128, word << 16, word & HIMASK)
        return plsc.bitcast(bits, jnp.float32)

    def make_group(c):
        def group(j, carry):
            pos_acc, neg_acc = carry
            g = c * GPC + j
            rows = lax.iota(jnp.int32, 16) + g * 16
            lab = lab_v[pl.ds(g * 16, 16)]
            dpos = unpack(plsc.load_gather(s_v, [rows, lab & 127]), lab)
            pos_val = jnp.maximum(dpos - POS_THRESH, 0.0)
            pos_v[pl.ds(g * 16, 16)] = pos_val
            acc = jnp.zeros((16,), jnp.float32)
            nb = (c * GPC * 16) // 128
            noff = (c * GPC * 16) % 128
            for k in range(NUM_NEG):
                nk = neg_v[k, nb, pl.ds(noff + j * 16, 16)]
                acc = acc + unpack(
                    plsc.load_gather(s_v, [rows, nk & 127]), nk)
            neg_val = jnp.maximum(NEG_THRESH - acc * (1.0 / NUM_NEG), 0.0)
            nout_v[pl.ds(g * 16, 16)] = neg_val
            return pos_acc + pos_val, neg_acc + neg_val
        return group

    pos_sum = jnp.zeros((16,), jnp.float32)
    neg_sum = jnp.zeros((16,), jnp.float32)
    for c in range(CHUNKS):
        copies[c].wait()
        pos_sum, neg_sum = lax.fori_loop(
            0, GPC, make_group(c), (pos_sum, neg_sum))

    psum_v[pl.ds(0, 16)] = pos_sum
    psum_v[pl.ds(16, 16)] = neg_sum
    pltpu.sync_copy(pos_v, pos_hbm.at[pl.ds(base, ROWS)])
    pltpu.sync_copy(nout_v, negout_hbm.at[pl.ds(base, ROWS)])
    pltpu.sync_copy(psum_v, part_hbm.at[wid])


@jax.jit
def _run(features, labels, anchor_feats, neg_inds):
    packed = pl.pallas_call(
        _dist_body,
        grid=(GRID,),
        in_specs=[
            pl.BlockSpec((BLK, FEAT_DIM), lambda i: (i, 0)),
            pl.BlockSpec((NUM_LABELS, FEAT_DIM), lambda i: (0, 0)),
        ],
        out_specs=pl.BlockSpec((BLK, 128), lambda i: (i, 0)),
        out_shape=jax.ShapeDtypeStruct((N_POINTS, 128), jnp.int32),
    )(features, anchor_feats)
    negl = neg_inds.T.reshape(NUM_NEG, N_POINTS // 128, 128)
    pos, neg, part = _sc_gather(packed, labels, negl)
    loss = part.sum() * (1.0 / N_POINTS)
    return (loss, pos, neg)


def kernel(features, labels, anchor_feats, neg_inds):
    return _run(features, labels, anchor_feats, neg_inds)


# speedup vs baseline: 1.2542x; 1.0074x over previous
import functools

import jax
import jax.numpy as jnp
from jax import lax
from jax.experimental import pallas as pl
from jax.experimental.pallas import tpu as pltpu
from jax.experimental.pallas import tpu_sc as plsc

N_POINTS = 8192
FEAT_DIM = 512
NUM_LABELS = 200
NUM_NEG = 32

POS_THRESH = 0.1
NEG_THRESH = 0.5
EPS = 1e-07

BLK = 2048
GRID = N_POINTS // BLK

NW = 32
ROWS = N_POINTS // NW
GROUPS = ROWS // 16
CHUNKS = 4
CROWS = ROWS // CHUNKS
GPC = GROUPS // CHUNKS
NBLK = ROWS // 128
HIMASK = -65536
RND = 32768


def _dist_body(x_ref, a_ref, s_ref):
    f = x_ref[...]
    a = a_ref[...]
    g = lax.dot_general(f, a, (((1,), (1,)), ((), ())),
                        preferred_element_type=jnp.float32)
    fn = jnp.sum(f * f, axis=1, keepdims=True)
    ones = jnp.ones((1, FEAT_DIM), jnp.float32)
    an = lax.dot_general(ones, a * a, (((1,), (1,)), ((), ())),
                         preferred_element_type=jnp.float32)
    d2 = jnp.maximum(fn + an - 2.0 * g, 0.0)
    s = jnp.sqrt(d2 + EPS)
    lo = lax.shift_right_logical(
        lax.bitcast_convert_type(s[:, :128], jnp.int32) + RND, 16)
    s_ref[...] = lo
    hi = (lax.bitcast_convert_type(s[:, 128:], jnp.int32) + RND) & HIMASK
    s_ref[:, :72] = s_ref[:, :72] | hi


_sc_mesh = plsc.VectorSubcoreMesh(core_axis_name="c", subcore_axis_name="s")


@functools.partial(
    pl.kernel,
    mesh=_sc_mesh,
    compiler_params=pltpu.CompilerParams(
        use_tc_tiling_on_sc=False, needs_layout_passes=False),
    out_type=[
        jax.ShapeDtypeStruct((N_POINTS,), jnp.float32),
        jax.ShapeDtypeStruct((N_POINTS,), jnp.float32),
        jax.ShapeDtypeStruct((NW * 32,), jnp.float32),
    ],
    scratch_types=[
        pltpu.VMEM((ROWS, 128), jnp.int32),
        pltpu.VMEM((ROWS,), jnp.int32),
        pltpu.VMEM((NUM_NEG, NBLK, 128), jnp.int32),
        pltpu.VMEM((ROWS,), jnp.float32),
        pltpu.VMEM((ROWS,), jnp.float32),
        pltpu.VMEM((32,), jnp.float32),
        [pltpu.SemaphoreType.DMA] * (CHUNKS + 2),
    ],
)
def _sc_gather(s_hbm, lab_hbm, negt_hbm, pos_hbm, negout_hbm,
               part_hbm, s_v, lab_v, neg_v, pos_v, nout_v, psum_v, sems):
    wid = lax.axis_index("s") * 2 + lax.axis_index("c")
    base = wid * ROWS
    lab_cp = pltpu.async_copy(
        lab_hbm.at[pl.ds(base, ROWS)], lab_v, sems[CHUNKS])
    neg_cp = pltpu.async_copy(
        negt_hbm.at[:, pl.ds(wid * NBLK, NBLK), :], neg_v, sems[CHUNKS + 1])
    copies = [
        pltpu.async_copy(
            s_hbm.at[pl.ds(base + c * CROWS, CROWS)],
            s_v.at[pl.ds(c * CROWS, CROWS)],
            sems[c])
        for c in range(CHUNKS)
    ]
    lab_cp.wait()
    neg_cp.wait()

    def unpack(word, col):
        bits = jnp.where(col < 128, word << 16, word & HIMASK)
        return plsc.bitcast(bits, jnp.float32)

    def make_group(c):
        def group(j, carry):
            pos_acc, neg_acc = carry
            g = c * GPC + j
            rows = lax.iota(jnp.int32, 16) + g * 16
            lab = lab_v[pl.ds(g * 16, 16)]
            dpos = unpack(plsc.load_gather(s_v, [rows, lab & 127]), lab)
            pos_val = jnp.maximum(dpos - POS_THRESH, 0.0)
            pos_v[pl.ds(g * 16, 16)] = pos_val
            acc = jnp.zeros((16,), jnp.float32)
            nb = (c * GPC * 16) // 128
            noff = (c * GPC * 16) % 128
            for k in range(NUM_NEG):
                nk = neg_v[k, nb, pl.ds(noff + j * 16, 16)]
                acc = acc + unpack(
                    plsc.load_gather(s_v, [rows, nk & 127]), nk)
            neg_val = jnp.maximum(NEG_THRESH - acc * (1.0 / NUM_NEG), 0.0)
            nout_v[pl.ds(g * 16, 16)] = neg_val
            return pos_acc + pos_val, neg_acc + neg_val
        return group

    pos_sum = jnp.zeros((16,), jnp.float32)
    neg_sum = jnp.zeros((16,), jnp.float32)
    for c in range(CHUNKS):
        copies[c].wait()
        pos_sum, neg_sum = lax.fori_loop(
            0, GPC, make_group(c), (pos_sum, neg_sum))

    psum_v[pl.ds(0, 16)] = pos_sum
    psum_v[pl.ds(16, 16)] = neg_sum
    pltpu.sync_copy(pos_v, pos_hbm.at[pl.ds(base, ROWS)])
    pltpu.sync_copy(nout_v, negout_hbm.at[pl.ds(base, ROWS)])
    pltpu.sync_copy(psum_v, part_hbm.at[pl.ds(wid * 32, 32)])


@jax.jit
def _run(features, labels, anchor_feats, neg_inds):
    packed = pl.pallas_call(
        _dist_body,
        grid=(GRID,),
        in_specs=[
            pl.BlockSpec((BLK, FEAT_DIM), lambda i: (i, 0)),
            pl.BlockSpec((NUM_LABELS, FEAT_DIM), lambda i: (0, 0)),
        ],
        out_specs=pl.BlockSpec((BLK, 128), lambda i: (i, 0)),
        out_shape=jax.ShapeDtypeStruct((N_POINTS, 128), jnp.int32),
    )(features, anchor_feats)
    negl = neg_inds.T.reshape(NUM_NEG, N_POINTS // 128, 128)
    pos, neg, part = _sc_gather(packed, labels, negl)
    loss = part.sum() * (1.0 / N_POINTS)
    return (loss, pos, neg)


def kernel(features, labels, anchor_feats, neg_inds):
    return _run(features, labels, anchor_feats, neg_inds)
